# Initial kernel scaffold; baseline (speedup 1.0000x reference)
import jax
import jax.numpy as jnp
from jax.experimental import pallas as pl


def kernel(features, idx):
    raise NotImplementedError("write your pallas kernel here")


# speedup vs baseline: 1.7718x; 1.7718x over previous
import functools

import jax
import jax.numpy as jnp
from jax import lax
from jax.experimental import pallas as pl
from jax.experimental.pallas import tpu as pltpu
from jax.experimental.pallas import tpu_sc as plsc


def _tr_body(x_ref, o_ref):
    o_ref[0] = x_ref[0].T


def _transpose_minor2(x, tn):
    b, p, q = x.shape
    return pl.pallas_call(
        _tr_body,
        grid=(b, q // tn),
        in_specs=[pl.BlockSpec((1, p, tn), lambda i, j: (i, 0, j))],
        out_specs=pl.BlockSpec((1, tn, p), lambda i, j: (i, j, 0)),
        out_shape=jax.ShapeDtypeStruct((b, q, p), x.dtype),
    )(x)


def _make_sc_gather(total_rows, table_rows_per_batch, c, rows_per_batch):
    info = plsc.get_sparse_core_info()
    nc, ns = info.num_cores, info.num_subcores
    nw = nc * ns
    per_w = total_rows // nw
    chunk = 128
    n_chunks = per_w // chunk

    @functools.partial(
        pl.kernel,
        out_type=jax.ShapeDtypeStruct((total_rows, c), jnp.float32),
        mesh=plsc.VectorSubcoreMesh(core_axis_name="c", subcore_axis_name="s"),
        scratch_types=[
            pltpu.VMEM((chunk,), jnp.int32),
            pltpu.VMEM((chunk, c), jnp.float32),
            pltpu.SemaphoreType.DMA,
        ],
    )
    def gather(table_hbm, idx_hbm, out_hbm, idx_v, rows_v, sem):
        wid = lax.axis_index("s") * nc + lax.axis_index("c")
        base = wid * per_w
        batch = base // rows_per_batch
        row_off = batch * table_rows_per_batch
        for k in range(n_chunks):
            start = base + k * chunk
            pltpu.sync_copy(idx_hbm.at[pl.ds(start, chunk)], idx_v)
            for i in range(chunk // 16):
                sl = pl.ds(i * 16, 16)
                idx_v[sl] = idx_v[sl] + row_off
            pltpu.async_copy(table_hbm.at[idx_v], rows_v, sem).wait()
            pltpu.sync_copy(rows_v, out_hbm.at[pl.ds(start, chunk)])

    return gather


def kernel(features, idx):
    b, c, n = features.shape
    m = idx.shape[1]
    ft = _transpose_minor2(features, tn=512)
    gather = _make_sc_gather(b * m, n, c, m)
    out_t = gather(ft.reshape(b * n, c), idx.reshape(b * m))
    return _transpose_minor2(out_t.reshape(b, m, c), tn=512)


# speedup vs baseline: 2.9242x; 1.6504x over previous
import functools

import jax
import jax.numpy as jnp
from jax import lax
from jax.experimental import pallas as pl
from jax.experimental.pallas import tpu as pltpu
from jax.experimental.pallas import tpu_sc as plsc


def _tr_body(x_ref, o_ref):
    o_ref[0] = x_ref[0].T


def _transpose_minor2(x, tn):
    b, p, q = x.shape
    return pl.pallas_call(
        _tr_body,
        grid=(b, q // tn),
        in_specs=[pl.BlockSpec((1, p, tn), lambda i, j: (i, 0, j))],
        out_specs=pl.BlockSpec((1, tn, p), lambda i, j: (i, j, 0)),
        out_shape=jax.ShapeDtypeStruct((b, q, p), x.dtype),
    )(x)


def _make_sc_gather(total_rows, table_rows_per_batch, c, rows_per_batch):
    info = plsc.get_sparse_core_info()
    nc, ns = info.num_cores, info.num_subcores
    nw = nc * ns
    per_w = total_rows // nw
    chunk = 128
    n_chunks = per_w // chunk

    @functools.partial(
        pl.kernel,
        out_type=jax.ShapeDtypeStruct((total_rows, c), jnp.float32),
        mesh=plsc.VectorSubcoreMesh(core_axis_name="c", subcore_axis_name="s"),
        scratch_types=[
            pltpu.VMEM((chunk,), jnp.int32),
            pltpu.VMEM((chunk, c), jnp.float32),
            pltpu.SemaphoreType.DMA,
        ],
    )
    def gather(table_hbm, idx_hbm, out_hbm, idx_v, rows_v, sem):
        wid = lax.axis_index("s") * nc + lax.axis_index("c")
        base = wid * per_w
        batch = base // rows_per_batch
        row_off = batch * table_rows_per_batch
        for k in range(n_chunks):
            start = base + k * chunk
            pltpu.sync_copy(idx_hbm.at[pl.ds(start, chunk)], idx_v)
            for i in range(chunk // 16):
                sl = pl.ds(i * 16, 16)
                idx_v[sl] = idx_v[sl] + row_off
            pltpu.async_copy(table_hbm.at[idx_v], rows_v, sem).wait()
            pltpu.sync_copy(rows_v, out_hbm.at[pl.ds(start, chunk)])

    return gather


def kernel(features, idx):
    b, c, n = features.shape
    m = idx.shape[1]
    ft = _transpose_minor2(features, tn=2048)
    gather = _make_sc_gather(b * m, n, c, m)
    out_t = gather(ft.reshape(b * n, c), idx.reshape(b * m))
    return _transpose_minor2(out_t.reshape(b, m, c), tn=1024)
